# baseline (speedup 1.0000x reference)
import functools

import jax
import jax.numpy as jnp
from jax import lax
from jax.experimental import pallas as pl
from jax.experimental.pallas import tpu as pltpu
from jax.experimental.pallas import tpu_sc as plsc

GX, GY, GZ, D = 64, 64, 64, 32
V = GX * GY * GZ
N = 262144

NC, NS, L = 2, 16, 16
NW = NC * NS
BPW = N // NW
C = 2048
NCH = BPW // C

_mesh = plsc.VectorSubcoreMesh(core_axis_name="c", subcore_axis_name="s")


@functools.partial(
    pl.kernel,
    mesh=_mesh,
    out_type=jax.ShapeDtypeStruct((N, D), jnp.float32),
    compiler_params=pltpu.CompilerParams(
        needs_layout_passes=False, use_tc_tiling_on_sc=False
    ),
    scratch_types=[
        pltpu.VMEM((C * 3,), jnp.int32),
        pltpu.VMEM((C,), jnp.int32),
        pltpu.VMEM((C, D), jnp.float32),
        pltpu.SemaphoreType.DMA,
    ],
)
def _sc_gather(table_hbm, idx_hbm, out_hbm, idx_v, lin_v, rows_v, sem):
    wid = lax.axis_index("s") * NC + lax.axis_index("c")
    base = wid * BPW

    def chunk(g, carry):
        off = base + g * C
        pltpu.sync_copy(idx_hbm.at[pl.ds(off, C)], idx_v.at[pl.ds(0, C)])
        pltpu.sync_copy(idx_hbm.at[pl.ds(N + off, C)], idx_v.at[pl.ds(C, C)])
        pltpu.sync_copy(idx_hbm.at[pl.ds(2 * N + off, C)], idx_v.at[pl.ds(2 * C, C)])

        def linearize(i, carry2):
            s = pl.ds(i * L, L)
            x = idx_v[pl.ds(i * L, L)]
            y = idx_v[pl.ds(C + i * L, L)]
            z = idx_v[pl.ds(2 * C + i * L, L)]
            lin_v[s] = (x << 12) + (y << 6) + z
            return carry2

        lax.fori_loop(0, C // L, linearize, 0, unroll=4)

        pltpu.async_copy(table_hbm.at[lin_v], rows_v, sem).wait()
        pltpu.sync_copy(rows_v, out_hbm.at[pl.ds(off, C)])
        return carry

    lax.fori_loop(0, NCH, chunk, 0)


def _tc_detile_body(t2_ref, out_ref):
    x = t2_ref[...]
    y = jnp.transpose(x, (0, 2, 1))
    z = y.reshape(16, 16, 4, 32)
    out_ref[...] = jnp.concatenate(
        [z[:, :, 0, :], z[:, :, 1, :], z[:, :, 2, :], z[:, :, 3, :]], axis=2
    )


_tc_detile = pl.pallas_call(
    _tc_detile_body,
    grid=(256,),
    in_specs=[pl.BlockSpec((16, 32, 64), lambda i: (i, 0, 0))],
    out_specs=pl.BlockSpec((16, 16, 128), lambda i: (i, 0, 0)),
    out_shape=jax.ShapeDtypeStruct((4096, 16, 128), jnp.float32),
)


def kernel(in_tensor, in_index):
    t2 = jnp.transpose(in_tensor, (0, 1, 3, 2)).reshape(4096, 32, 64)
    table = _tc_detile(t2).reshape(V, D)
    idx = in_index.astype(jnp.int32).T.reshape(3 * N)
    return _sc_gather(table, idx)


# speedup vs baseline: 1.5222x; 1.5222x over previous
import functools

import jax
import jax.numpy as jnp
from jax import lax
from jax.experimental import pallas as pl
from jax.experimental.pallas import tpu as pltpu
from jax.experimental.pallas import tpu_sc as plsc

GX, GY, GZ, D = 64, 64, 64, 32
V = GX * GY * GZ
N = 262144

NC, NS, L = 2, 16, 16
NW = NC * NS
BPW = N // NW
C = 2048
NCH = BPW // C

_mesh = plsc.VectorSubcoreMesh(core_axis_name="c", subcore_axis_name="s")


@functools.partial(
    pl.kernel,
    mesh=_mesh,
    out_type=jax.ShapeDtypeStruct((N, D), jnp.float32),
    compiler_params=pltpu.CompilerParams(
        needs_layout_passes=False, use_tc_tiling_on_sc=False
    ),
    scratch_types=[
        pltpu.VMEM((C * 3,), jnp.int32),
        pltpu.VMEM((C,), jnp.int32),
        pltpu.VMEM((C, D), jnp.float32),
        pltpu.SemaphoreType.DMA,
    ],
)
def _sc_gather(table_hbm, idx_hbm, out_hbm, idx_v, lin_v, rows_v, sem):
    wid = lax.axis_index("s") * NC + lax.axis_index("c")
    base = wid * BPW

    def chunk(g, carry):
        off = base + g * C
        pltpu.sync_copy(idx_hbm.at[pl.ds(off, C)], idx_v.at[pl.ds(0, C)])
        pltpu.sync_copy(idx_hbm.at[pl.ds(N + off, C)], idx_v.at[pl.ds(C, C)])
        pltpu.sync_copy(idx_hbm.at[pl.ds(2 * N + off, C)], idx_v.at[pl.ds(2 * C, C)])

        def linearize(i, carry2):
            sl = pl.ds(i * L, L)
            x = idx_v[pl.ds(i * L, L)]
            y = idx_v[pl.ds(C + i * L, L)]
            z = idx_v[pl.ds(2 * C + i * L, L)]
            s = (x << 6) + y
            lin_v[sl] = ((s >> 2) << 8) + (z << 2) + (s & 3)
            return carry2

        lax.fori_loop(0, C // L, linearize, 0, unroll=4)

        pltpu.async_copy(table_hbm.at[lin_v], rows_v, sem).wait()
        pltpu.sync_copy(rows_v, out_hbm.at[pl.ds(off, C)])
        return carry

    lax.fori_loop(0, NCH, chunk, 0)


def _tc_detile_body(t2_ref, out_ref):
    x = t2_ref[...]
    xr = x.reshape(16, 128, 64)
    out_ref[...] = jnp.transpose(xr, (0, 2, 1))


_tc_detile = pl.pallas_call(
    _tc_detile_body,
    grid=(64,),
    in_specs=[pl.BlockSpec((64, 32, 64), lambda i: (i, 0, 0))],
    out_specs=pl.BlockSpec((16, 64, 128), lambda i: (i, 0, 0)),
    out_shape=jax.ShapeDtypeStruct((1024, 64, 128), jnp.float32),
)


def kernel(in_tensor, in_index):
    t2 = jnp.transpose(in_tensor, (0, 1, 3, 2)).reshape(4096, 32, 64)
    table = _tc_detile(t2).reshape(V, D)
    idx = in_index.astype(jnp.int32).T.reshape(3 * N)
    return _sc_gather(table, idx)
